# baseline (speedup 1.0000x reference)
import functools

import jax
import jax.numpy as jnp
from jax import lax
from jax.experimental import pallas as pl
from jax.experimental.pallas import tpu as pltpu
from jax.experimental.pallas import tpu_sc as plsc

_TOPK = 100
_N_REL = 999000
_N_CLS = 51
_GRID = 122
_R = 8192

_NC = 2
_NS = 16
_L = 16
_CH = 31232
_PAD_N = _NC * _NS * _CH
_SUB = 7808
_NSUBCH = _CH // _SUB
_VPS = _SUB // _L
_VCH = _CH // _L
_NBIN = 4096
_BIN_SHIFT = 18
_CAPT = 512
_CAPTP = _CAPT + _L
_SLOTS = _NS * _CAPTP
_STATIC_CAND = 2048
_TOP = 112


def _al(x):
    return pl.multiple_of(x, _L)


def _maxarg_body(x_ref, prob_ref, label_ref):
    x = x_ref[...]
    xt = x.T
    prob = jnp.maximum(jnp.max(xt[1:, :], axis=0), 0.0)
    eq = (xt[1:, :] == prob[None, :]).astype(jnp.float32)
    wexp = (126 - jax.lax.iota(jnp.int32, _N_CLS - 1)) << 23
    w = jax.lax.bitcast_convert_type(wexp, jnp.float32)
    d = jax.lax.dot_general(w[None, :], eq, (((1,), (0,)), ((), ())),
                            preferred_element_type=jnp.float32)
    dbits = jax.lax.bitcast_convert_type(d[0], jnp.int32)
    lab = 127 - (dbits >> 23)
    label = jnp.where(prob > 0.0, lab, 0)
    prob_ref[...] = prob
    label_ref[...] = label


def _maxarg(rel_det_prob):
    prob, label = pl.pallas_call(
        _maxarg_body,
        grid=(_GRID,),
        in_specs=[pl.BlockSpec((_R, _N_CLS), lambda i: (i, 0))],
        out_specs=[
            pl.BlockSpec((_R,), lambda i: (i,)),
            pl.BlockSpec((_R,), lambda i: (i,)),
        ],
        out_shape=[
            jax.ShapeDtypeStruct((_PAD_N,), jnp.float32),
            jax.ShapeDtypeStruct((_PAD_N,), jnp.int32),
        ],
    )(rel_det_prob)
    return prob, label


def _sc_mesh():
    return plsc.VectorSubcoreMesh(core_axis_name="c", subcore_axis_name="s")


def _select_body(prob_hbm, sub_hbm, obj_hbm, scores_hbm, oval_hbm, oidx_hbm,
                 scores_v, pv, sv, ov, score_buf, hist_v, cand_v, cand_i,
                 all_v, all_i, rank_v, counts_v, rowidx_v, tmp16f, tmp16i,
                 out_val_v, out_idx_v,
                 sh_counts, sh_hist, sh_cval, sh_cidx, sh_rank):
    c = lax.axis_index("c")
    s = lax.axis_index("s")
    base = (c * _NS + s) * _CH
    lane = lax.iota(jnp.int32, _L)
    ones = jnp.ones((_L,), jnp.int32)

    def _zero_hist(j, _):
        hist_v[j // 8, pl.ds(_al((j % 8) * _L), _L)] = jnp.zeros((_L,), jnp.int32)
        return 0
    lax.fori_loop(0, _NBIN // _L, _zero_hist, 0)

    def _fill_cand(j, _):
        cand_v[pl.ds(_al(j * _L), _L)] = jnp.full((_L,), -1.0, jnp.float32)
        cand_i[pl.ds(_al(j * _L), _L)] = jnp.zeros((_L,), jnp.int32)
        return 0
    lax.fori_loop(0, _CAPTP // _L, _fill_cand, 0)

    pltpu.sync_copy(scores_hbm, scores_v)

    for k in range(_NSUBCH):
        off = k * _SUB
        pltpu.sync_copy(prob_hbm.at[pl.ds(_al(base + off), _SUB)], pv)
        pltpu.sync_copy(sub_hbm.at[pl.ds(_al(base + off), _SUB)], sv)
        pltpu.sync_copy(obj_hbm.at[pl.ds(_al(base + off), _SUB)], ov)

        def _score(i, _, off=off):
            p16 = pv[pl.ds(_al(i * _L), _L)]
            s16 = sv[pl.ds(_al(i * _L), _L)]
            o16 = ov[pl.ds(_al(i * _L), _L)]
            ss = plsc.load_gather(scores_v, [s16])
            so = plsc.load_gather(scores_v, [o16])
            sc = p16 * ss * so
            score_buf[pl.ds(_al(off + i * _L), _L)] = sc
            gidx = base + off + i * _L + lane
            valid = gidx < _N_REL
            bins = lax.shift_right_logical(plsc.bitcast(sc, jnp.int32),
                                           _BIN_SHIFT)
            plsc.addupdate_scatter(hist_v, [bins >> 7, bins & 127], ones,
                                   mask=valid)
            return 0
        lax.fori_loop(0, _VPS, _score, 0)

    @pl.when(s == 0)
    def _():
        pltpu.sync_copy(hist_v, sh_hist)
    plsc.subcore_barrier()

    rowidx_v[pl.ds(0, _L)] = lane
    rowidx_v[pl.ds(_L, _L)] = lane + _L

    @pl.when(s != 0)
    def _():
        pltpu.sync_copy(hist_v, sh_hist.at[rowidx_v], add=True)
    plsc.subcore_barrier()
    pltpu.sync_copy(sh_hist, hist_v)

    def _thresh(j, carry):
        total, bbin = carry
        bin0 = _NBIN - _L * (j + 1)
        v = hist_v[bin0 // 128, pl.ds(_al(bin0 % 128), _L)]
        rv = lax.rev(v, (0,))
        cs = plsc.cumsum(rv) + total
        hit = cs >= _TOPK
        nhit = plsc.all_reduce_population_count(hit)[0]
        pos = plsc.all_reduce_ffs(hit)[0]
        newb = bin0 + (_L - 1) - pos
        bbin = jnp.where((bbin < 0) & (nhit > 0), newb, bbin)
        return cs[_L - 1], bbin
    _, bbin = lax.fori_loop(0, _NBIN // _L, _thresh,
                            (jnp.int32(0), jnp.int32(-1)))
    bbin = jnp.maximum(bbin, 0)
    tbits = bbin << _BIN_SHIFT

    def _compact(i, cursor):
        sc = score_buf[pl.ds(_al(i * _L), _L)]
        bits = plsc.bitcast(sc, jnp.int32)
        gidx = base + i * _L + lane
        m = (bits >= tbits) & (gidx < _N_REL)
        mi32 = m.astype(jnp.int32)
        cnt = plsc.all_reduce_population_count(m)[0]
        pos = jnp.minimum(cursor + plsc.cumsum(mi32) - 1, _CAPTP - 1)
        plsc.store_scatter(cand_v, [pos], sc, mask=m)
        plsc.store_scatter(cand_i, [pos], gidx, mask=m)
        return cursor + cnt
    n_s = lax.fori_loop(0, _VCH, _compact, jnp.int32(0))
    n_s = jnp.minimum(n_s, _CAPT)
    n_pad = ((n_s + _L - 1) // _L) * _L

    tmp16i[...] = jnp.broadcast_to(n_pad, (_L,)).astype(jnp.int32)
    pltpu.sync_copy(tmp16i, sh_counts.at[pl.ds(_al(s * _L), _L)])
    plsc.subcore_barrier()
    pltpu.sync_copy(sh_counts, counts_v)

    offset = jnp.int32(0)
    total = jnp.int32(0)
    for t in range(_NS):
        ct = counts_v[pl.ds(_al(t * _L), _L)][0]
        offset = jnp.where(t < s, offset + ct, offset)
        total = total + ct

    def _pub(t, _):
        pltpu.sync_copy(cand_v.at[pl.ds(_al(t * _L), _L)],
                        sh_cval.at[pl.ds(_al(offset + t * _L), _L)])
        pltpu.sync_copy(cand_i.at[pl.ds(_al(t * _L), _L)],
                        sh_cidx.at[pl.ds(_al(offset + t * _L), _L)])
        return 0
    lax.fori_loop(0, n_pad // _L, _pub, 0)
    plsc.subcore_barrier()

    nv = total // _L
    pltpu.sync_copy(sh_cval.at[pl.ds(0, _STATIC_CAND)],
                    all_v.at[pl.ds(0, _STATIC_CAND)])
    pltpu.sync_copy(sh_cidx.at[pl.ds(0, _STATIC_CAND)],
                    all_i.at[pl.ds(0, _STATIC_CAND)])

    @pl.when(total > _STATIC_CAND)
    def _():
        def _extra(t, _):
            w = _STATIC_CAND + t * _L
            pltpu.sync_copy(sh_cval.at[pl.ds(_al(w), _L)], all_v.at[pl.ds(_al(w), _L)])
            pltpu.sync_copy(sh_cidx.at[pl.ds(_al(w), _L)], all_i.at[pl.ds(_al(w), _L)])
            return 0
        lax.fori_loop(0, (total - _STATIC_CAND) // _L, _extra, 0)

    na = jnp.maximum(0, (nv - s + _NS - 1) // _NS)

    def _rank_one(t, _):
        a = s + t * _NS
        va = all_v[pl.ds(_al(a * _L), _L)]
        ia = all_i[pl.ds(_al(a * _L), _L)]

        def _against(b, r):
            vb = all_v[pl.ds(_al(b * _L), _L)]
            ib = all_i[pl.ds(_al(b * _L), _L)]
            for q in range(_L):
                bv = jnp.full((_L,), vb[q], jnp.float32)
                bi = jnp.full((_L,), ib[q], jnp.int32)
                beats = (bv > va) | ((bv == va) & (bi < ia))
                r = r + beats.astype(jnp.int32)
            return r
        rank = lax.fori_loop(0, nv, _against, jnp.zeros((_L,), jnp.int32))
        tmp16i[...] = rank
        pltpu.sync_copy(tmp16i, sh_rank.at[pl.ds(_al(a * _L), _L)])
        return 0
    lax.fori_loop(0, na, _rank_one, 0)
    plsc.subcore_barrier()

    @pl.when(s == 0)
    def _():
        for j in range(_TOP // _L):
            out_val_v[pl.ds(_al(j * _L), _L)] = jnp.full((_L,), -1.0, jnp.float32)
            out_idx_v[pl.ds(_al(j * _L), _L)] = jnp.zeros((_L,), jnp.int32)
        pltpu.sync_copy(sh_rank.at[pl.ds(0, _STATIC_CAND)],
                        rank_v.at[pl.ds(0, _STATIC_CAND)])

        @pl.when(total > _STATIC_CAND)
        def _():
            def _extra_r(t, _):
                w = _STATIC_CAND + t * _L
                pltpu.sync_copy(sh_rank.at[pl.ds(_al(w), _L)],
                                rank_v.at[pl.ds(_al(w), _L)])
                return 0
            lax.fori_loop(0, (total - _STATIC_CAND) // _L, _extra_r, 0)

        def _emit(t, _):
            rv = rank_v[pl.ds(_al(t * _L), _L)]
            m = rv < _TOPK
            ridx = jnp.minimum(rv, _TOP - 1)
            plsc.store_scatter(out_val_v, [ridx],
                               all_v[pl.ds(_al(t * _L), _L)], mask=m)
            plsc.store_scatter(out_idx_v, [ridx],
                               all_i[pl.ds(_al(t * _L), _L)], mask=m)
            return 0
        lax.fori_loop(0, nv, _emit, 0)
        pltpu.sync_copy(out_val_v, oval_hbm.at[pl.ds(_al(c * _TOP), _TOP)])
        pltpu.sync_copy(out_idx_v, oidx_hbm.at[pl.ds(_al(c * _TOP), _TOP)])


def _sc_select(prob_p, sub_p, obj_p, scores_p):
    fn = functools.partial(
        pl.kernel,
        compiler_params=pltpu.CompilerParams(needs_layout_passes=False),
        out_type=[
            jax.ShapeDtypeStruct((_NC * _TOP,), jnp.float32),
            jax.ShapeDtypeStruct((_NC * _TOP,), jnp.int32),
        ],
        mesh=_sc_mesh(),
        scratch_types=[
            pltpu.VMEM((1024,), jnp.float32),
            pltpu.VMEM((_SUB,), jnp.float32),
            pltpu.VMEM((_SUB,), jnp.int32),
            pltpu.VMEM((_SUB,), jnp.int32),
            pltpu.VMEM((_CH,), jnp.float32),
            pltpu.VMEM((2 * _L, _NBIN // (2 * _L)), jnp.int32),
            pltpu.VMEM((_CAPTP,), jnp.float32),
            pltpu.VMEM((_CAPTP,), jnp.int32),
            pltpu.VMEM((_SLOTS,), jnp.float32),
            pltpu.VMEM((_SLOTS,), jnp.int32),
            pltpu.VMEM((_SLOTS,), jnp.int32),
            pltpu.VMEM((_NS * _L,), jnp.int32),
            pltpu.VMEM((2 * _L,), jnp.int32),
            pltpu.VMEM((_L,), jnp.float32),
            pltpu.VMEM((_L,), jnp.int32),
            pltpu.VMEM((_TOP,), jnp.float32),
            pltpu.VMEM((_TOP,), jnp.int32),
            pltpu.VMEM_SHARED((_NS * _L,), jnp.int32),
            pltpu.VMEM_SHARED((2 * _L, _NBIN // (2 * _L)), jnp.int32),
            pltpu.VMEM_SHARED((_SLOTS,), jnp.float32),
            pltpu.VMEM_SHARED((_SLOTS,), jnp.int32),
            pltpu.VMEM_SHARED((_SLOTS,), jnp.int32),
        ],
    )
    return fn(_select_body)(prob_p, sub_p, obj_p, scores_p)


def _merge_body(oval_hbm, oidx_hbm, prob_hbm, label_hbm, sub_hbm, obj_hbm,
                conn_hbm, lab_hbm, prb_hbm,
                val_v, idx_v, mv, mi, g1, g2, g3, g4, conn_v, sem):
    c = lax.axis_index("c")
    s = lax.axis_index("s")

    @pl.when((c == 0) & (s == 0))
    def _():
        lane = lax.iota(jnp.int32, _L)
        pltpu.sync_copy(oval_hbm, val_v)
        pltpu.sync_copy(oidx_hbm, idx_v)
        for j in range(_TOP // _L):
            mv[pl.ds(j * _L, _L)] = jnp.full((_L,), -1.0, jnp.float32)
            mi[pl.ds(j * _L, _L)] = jnp.zeros((_L,), jnp.int32)

        for side in range(2):
            for j in range(_TOP // _L):
                pos = j * _L + lane
                va = val_v[pl.ds(side * _TOP + j * _L, _L)]
                ia = idx_v[pl.ds(side * _TOP + j * _L, _L)]
                cnt = jnp.zeros((_L,), jnp.int32)
                for jb in range(_TOP // _L):
                    vb = val_v[pl.ds((1 - side) * _TOP + jb * _L, _L)]
                    ib = idx_v[pl.ds((1 - side) * _TOP + jb * _L, _L)]
                    for q in range(_L):
                        bv = jnp.full((_L,), vb[q], jnp.float32)
                        bi = jnp.full((_L,), ib[q], jnp.int32)
                        beats = (bv > va) | ((bv == va) & (bi < ia))
                        cnt = cnt + beats.astype(jnp.int32)
                rank = pos + cnt
                m = rank < _TOPK
                ridx = jnp.minimum(rank, _TOP - 1)
                plsc.store_scatter(mv, [ridx], va, mask=m)
                plsc.store_scatter(mi, [ridx], ia, mask=m)

        pltpu.async_copy(prob_hbm.at[mi], g1, sem).wait()
        pltpu.async_copy(label_hbm.at[mi], g2, sem).wait()
        pltpu.async_copy(sub_hbm.at[mi], g3, sem).wait()
        pltpu.async_copy(obj_hbm.at[mi], g4, sem).wait()

        for j in range(_TOP // _L):
            pos2 = 2 * (j * _L + lane)
            sub16 = g3[pl.ds(j * _L, _L)]
            obj16 = g4[pl.ds(j * _L, _L)]
            plsc.store_scatter(conn_v, [pos2], sub16)
            plsc.store_scatter(conn_v, [pos2 + 1], obj16)

        pltpu.sync_copy(conn_v, conn_hbm)
        pltpu.sync_copy(g2, lab_hbm)
        pltpu.sync_copy(g1, prb_hbm)


def _sc_merge(oval, oidx, prob, label, sub, obj):
    fn = functools.partial(
        pl.kernel,
        compiler_params=pltpu.CompilerParams(needs_layout_passes=False),
        out_type=[
            jax.ShapeDtypeStruct((2 * _TOP,), jnp.int32),
            jax.ShapeDtypeStruct((_TOP,), jnp.int32),
            jax.ShapeDtypeStruct((_TOP,), jnp.float32),
        ],
        mesh=_sc_mesh(),
        scratch_types=[
            pltpu.VMEM((_NC * _TOP,), jnp.float32),
            pltpu.VMEM((_NC * _TOP,), jnp.int32),
            pltpu.VMEM((_TOP,), jnp.float32),
            pltpu.VMEM((_TOP,), jnp.int32),
            pltpu.VMEM((_TOP,), jnp.float32),
            pltpu.VMEM((_TOP,), jnp.int32),
            pltpu.VMEM((_TOP,), jnp.int32),
            pltpu.VMEM((_TOP,), jnp.int32),
            pltpu.VMEM((2 * _TOP,), jnp.int32),
            pltpu.SemaphoreType.DMA,
        ],
    )
    return fn(_merge_body)(oval, oidx, prob, label, sub, obj)


def kernel(rel_det_prob, scores, connect_arr):
    prob_p, label_p = _maxarg(rel_det_prob)
    pad = _PAD_N - _N_REL
    sub = connect_arr[0]
    obj = connect_arr[1]
    sub_p = jnp.concatenate([sub, jnp.zeros((pad,), jnp.int32)])
    obj_p = jnp.concatenate([obj, jnp.zeros((pad,), jnp.int32)])
    scores_p = jnp.concatenate([scores, jnp.zeros((24,), jnp.float32)])

    oval, oidx = _sc_select(prob_p, sub_p, obj_p, scores_p)
    conn_flat, lab2d, prb2d = _sc_merge(oval, oidx, prob_p, label_p, sub_p,
                                        obj_p)

    conn_sel = conn_flat[: 2 * _TOPK].reshape(_TOPK, 2)
    labels_sel = lab2d[:_TOPK]
    probs_sel = prb2d[:_TOPK]
    return conn_sel, labels_sel, probs_sel


# speedup vs baseline: 1.0565x; 1.0565x over previous
import functools

import jax
import jax.numpy as jnp
from jax import lax
from jax.experimental import pallas as pl
from jax.experimental.pallas import tpu as pltpu
from jax.experimental.pallas import tpu_sc as plsc

_TOPK = 100
_N_REL = 999000
_N_CLS = 51
_GRID = 61
_R = 16384

_NC = 2
_NS = 16
_L = 16
_CH = 31232
_PAD_N = _NC * _NS * _CH
_SUB = 7808
_NSUBCH = _CH // _SUB
_VPS = _SUB // _L
_VCH = _CH // _L
_NBIN = 4096
_BIN_SHIFT = 18
_CAPT = 512
_CAPTP = _CAPT + _L
_SLOTS = _NS * _CAPTP
_STATIC_CAND = 2048
_TOP = 112


def _al(x):
    return pl.multiple_of(x, _L)


def _maxarg_body(x_ref, prob_ref, label_ref):
    x = x_ref[...]
    xt = x.T
    prob = jnp.maximum(jnp.max(xt[1:, :], axis=0), 0.0)
    eq = (xt[1:, :] == prob[None, :]).astype(jnp.float32)
    wexp = (126 - jax.lax.iota(jnp.int32, _N_CLS - 1)) << 23
    w = jax.lax.bitcast_convert_type(wexp, jnp.float32)
    d = jax.lax.dot_general(w[None, :], eq, (((1,), (0,)), ((), ())),
                            preferred_element_type=jnp.float32)
    dbits = jax.lax.bitcast_convert_type(d[0], jnp.int32)
    lab = 127 - (dbits >> 23)
    label = jnp.where(prob > 0.0, lab, 0)
    prob_ref[...] = prob
    label_ref[...] = label


def _maxarg(rel_det_prob):
    prob, label = pl.pallas_call(
        _maxarg_body,
        grid=(_GRID,),
        in_specs=[pl.BlockSpec((_R, _N_CLS), lambda i: (i, 0))],
        out_specs=[
            pl.BlockSpec((_R,), lambda i: (i,)),
            pl.BlockSpec((_R,), lambda i: (i,)),
        ],
        out_shape=[
            jax.ShapeDtypeStruct((_PAD_N,), jnp.float32),
            jax.ShapeDtypeStruct((_PAD_N,), jnp.int32),
        ],
    )(rel_det_prob)
    return prob, label


def _sc_mesh():
    return plsc.VectorSubcoreMesh(core_axis_name="c", subcore_axis_name="s")


def _select_body(prob_hbm, sub_hbm, obj_hbm, scores_hbm, oval_hbm, oidx_hbm,
                 scores_v, pv, sv, ov, score_buf, hist_v, cand_v, cand_i,
                 all_v, all_i, rank_v, counts_v, rowidx_v, tmp16f, tmp16i,
                 out_val_v, out_idx_v,
                 sh_counts, sh_hist, sh_cval, sh_cidx, sh_rank):
    c = lax.axis_index("c")
    s = lax.axis_index("s")
    base = (c * _NS + s) * _CH
    lane = lax.iota(jnp.int32, _L)
    ones = jnp.ones((_L,), jnp.int32)

    def _zero_hist(j, _):
        hist_v[j // 8, pl.ds(_al((j % 8) * _L), _L)] = jnp.zeros((_L,), jnp.int32)
        return 0
    lax.fori_loop(0, _NBIN // _L, _zero_hist, 0)

    def _fill_cand(j, _):
        cand_v[pl.ds(_al(j * _L), _L)] = jnp.full((_L,), -1.0, jnp.float32)
        cand_i[pl.ds(_al(j * _L), _L)] = jnp.zeros((_L,), jnp.int32)
        return 0
    lax.fori_loop(0, _CAPTP // _L, _fill_cand, 0)

    pltpu.sync_copy(scores_hbm, scores_v)

    for k in range(_NSUBCH):
        off = k * _SUB
        pltpu.sync_copy(prob_hbm.at[pl.ds(_al(base + off), _SUB)], pv)
        pltpu.sync_copy(sub_hbm.at[pl.ds(_al(base + off), _SUB)], sv)
        pltpu.sync_copy(obj_hbm.at[pl.ds(_al(base + off), _SUB)], ov)

        def _score(i, _, off=off):
            p16 = pv[pl.ds(_al(i * _L), _L)]
            s16 = sv[pl.ds(_al(i * _L), _L)]
            o16 = ov[pl.ds(_al(i * _L), _L)]
            ss = plsc.load_gather(scores_v, [s16])
            so = plsc.load_gather(scores_v, [o16])
            sc = p16 * ss * so
            score_buf[pl.ds(_al(off + i * _L), _L)] = sc
            gidx = base + off + i * _L + lane
            valid = gidx < _N_REL
            bins = lax.shift_right_logical(plsc.bitcast(sc, jnp.int32),
                                           _BIN_SHIFT)
            plsc.addupdate_scatter(hist_v, [bins >> 7, bins & 127], ones,
                                   mask=valid)
            return 0
        lax.fori_loop(0, _VPS, _score, 0)

    @pl.when(s == 0)
    def _():
        pltpu.sync_copy(hist_v, sh_hist)
    plsc.subcore_barrier()

    rowidx_v[pl.ds(0, _L)] = lane
    rowidx_v[pl.ds(_L, _L)] = lane + _L

    @pl.when(s != 0)
    def _():
        pltpu.sync_copy(hist_v, sh_hist.at[rowidx_v], add=True)
    plsc.subcore_barrier()
    pltpu.sync_copy(sh_hist, hist_v)

    def _thresh(j, carry):
        total, bbin = carry
        bin0 = _NBIN - _L * (j + 1)
        v = hist_v[bin0 // 128, pl.ds(_al(bin0 % 128), _L)]
        rv = lax.rev(v, (0,))
        cs = plsc.cumsum(rv) + total
        hit = cs >= _TOPK
        nhit = plsc.all_reduce_population_count(hit)[0]
        pos = plsc.all_reduce_ffs(hit)[0]
        newb = bin0 + (_L - 1) - pos
        bbin = jnp.where((bbin < 0) & (nhit > 0), newb, bbin)
        return cs[_L - 1], bbin
    _, bbin = lax.fori_loop(0, _NBIN // _L, _thresh,
                            (jnp.int32(0), jnp.int32(-1)))
    bbin = jnp.maximum(bbin, 0)
    tbits = bbin << _BIN_SHIFT

    def _compact(i, cursor):
        sc = score_buf[pl.ds(_al(i * _L), _L)]
        bits = plsc.bitcast(sc, jnp.int32)
        gidx = base + i * _L + lane
        m = (bits >= tbits) & (gidx < _N_REL)
        mi32 = m.astype(jnp.int32)
        cnt = plsc.all_reduce_population_count(m)[0]
        pos = jnp.minimum(cursor + plsc.cumsum(mi32) - 1, _CAPTP - 1)
        plsc.store_scatter(cand_v, [pos], sc, mask=m)
        plsc.store_scatter(cand_i, [pos], gidx, mask=m)
        return cursor + cnt
    n_s = lax.fori_loop(0, _VCH, _compact, jnp.int32(0))
    n_s = jnp.minimum(n_s, _CAPT)
    n_pad = ((n_s + _L - 1) // _L) * _L

    tmp16i[...] = jnp.broadcast_to(n_pad, (_L,)).astype(jnp.int32)
    pltpu.sync_copy(tmp16i, sh_counts.at[pl.ds(_al(s * _L), _L)])
    plsc.subcore_barrier()
    pltpu.sync_copy(sh_counts, counts_v)

    offset = jnp.int32(0)
    total = jnp.int32(0)
    for t in range(_NS):
        ct = counts_v[pl.ds(_al(t * _L), _L)][0]
        offset = jnp.where(t < s, offset + ct, offset)
        total = total + ct

    def _pub(t, _):
        pltpu.sync_copy(cand_v.at[pl.ds(_al(t * _L), _L)],
                        sh_cval.at[pl.ds(_al(offset + t * _L), _L)])
        pltpu.sync_copy(cand_i.at[pl.ds(_al(t * _L), _L)],
                        sh_cidx.at[pl.ds(_al(offset + t * _L), _L)])
        return 0
    lax.fori_loop(0, n_pad // _L, _pub, 0)
    plsc.subcore_barrier()

    nv = total // _L
    pltpu.sync_copy(sh_cval.at[pl.ds(0, _STATIC_CAND)],
                    all_v.at[pl.ds(0, _STATIC_CAND)])
    pltpu.sync_copy(sh_cidx.at[pl.ds(0, _STATIC_CAND)],
                    all_i.at[pl.ds(0, _STATIC_CAND)])

    @pl.when(total > _STATIC_CAND)
    def _():
        def _extra(t, _):
            w = _STATIC_CAND + t * _L
            pltpu.sync_copy(sh_cval.at[pl.ds(_al(w), _L)], all_v.at[pl.ds(_al(w), _L)])
            pltpu.sync_copy(sh_cidx.at[pl.ds(_al(w), _L)], all_i.at[pl.ds(_al(w), _L)])
            return 0
        lax.fori_loop(0, (total - _STATIC_CAND) // _L, _extra, 0)

    na = jnp.maximum(0, (nv - s + _NS - 1) // _NS)

    def _rank_one(t, _):
        a = s + t * _NS
        va = all_v[pl.ds(_al(a * _L), _L)]
        ia = all_i[pl.ds(_al(a * _L), _L)]

        def _against(b, r):
            vb = all_v[pl.ds(_al(b * _L), _L)]
            ib = all_i[pl.ds(_al(b * _L), _L)]
            for q in range(_L):
                bv = jnp.full((_L,), vb[q], jnp.float32)
                bi = jnp.full((_L,), ib[q], jnp.int32)
                beats = (bv > va) | ((bv == va) & (bi < ia))
                r = r + beats.astype(jnp.int32)
            return r
        rank = lax.fori_loop(0, nv, _against, jnp.zeros((_L,), jnp.int32))
        tmp16i[...] = rank
        pltpu.sync_copy(tmp16i, sh_rank.at[pl.ds(_al(a * _L), _L)])
        return 0
    lax.fori_loop(0, na, _rank_one, 0)
    plsc.subcore_barrier()

    @pl.when(s == 0)
    def _():
        for j in range(_TOP // _L):
            out_val_v[pl.ds(_al(j * _L), _L)] = jnp.full((_L,), -1.0, jnp.float32)
            out_idx_v[pl.ds(_al(j * _L), _L)] = jnp.zeros((_L,), jnp.int32)
        pltpu.sync_copy(sh_rank.at[pl.ds(0, _STATIC_CAND)],
                        rank_v.at[pl.ds(0, _STATIC_CAND)])

        @pl.when(total > _STATIC_CAND)
        def _():
            def _extra_r(t, _):
                w = _STATIC_CAND + t * _L
                pltpu.sync_copy(sh_rank.at[pl.ds(_al(w), _L)],
                                rank_v.at[pl.ds(_al(w), _L)])
                return 0
            lax.fori_loop(0, (total - _STATIC_CAND) // _L, _extra_r, 0)

        def _emit(t, _):
            rv = rank_v[pl.ds(_al(t * _L), _L)]
            m = rv < _TOPK
            ridx = jnp.minimum(rv, _TOP - 1)
            plsc.store_scatter(out_val_v, [ridx],
                               all_v[pl.ds(_al(t * _L), _L)], mask=m)
            plsc.store_scatter(out_idx_v, [ridx],
                               all_i[pl.ds(_al(t * _L), _L)], mask=m)
            return 0
        lax.fori_loop(0, nv, _emit, 0)
        pltpu.sync_copy(out_val_v, oval_hbm.at[pl.ds(_al(c * _TOP), _TOP)])
        pltpu.sync_copy(out_idx_v, oidx_hbm.at[pl.ds(_al(c * _TOP), _TOP)])


def _sc_select(prob_p, sub_p, obj_p, scores_p):
    fn = functools.partial(
        pl.kernel,
        compiler_params=pltpu.CompilerParams(needs_layout_passes=False),
        out_type=[
            jax.ShapeDtypeStruct((_NC * _TOP,), jnp.float32),
            jax.ShapeDtypeStruct((_NC * _TOP,), jnp.int32),
        ],
        mesh=_sc_mesh(),
        scratch_types=[
            pltpu.VMEM((1024,), jnp.float32),
            pltpu.VMEM((_SUB,), jnp.float32),
            pltpu.VMEM((_SUB,), jnp.int32),
            pltpu.VMEM((_SUB,), jnp.int32),
            pltpu.VMEM((_CH,), jnp.float32),
            pltpu.VMEM((2 * _L, _NBIN // (2 * _L)), jnp.int32),
            pltpu.VMEM((_CAPTP,), jnp.float32),
            pltpu.VMEM((_CAPTP,), jnp.int32),
            pltpu.VMEM((_SLOTS,), jnp.float32),
            pltpu.VMEM((_SLOTS,), jnp.int32),
            pltpu.VMEM((_SLOTS,), jnp.int32),
            pltpu.VMEM((_NS * _L,), jnp.int32),
            pltpu.VMEM((2 * _L,), jnp.int32),
            pltpu.VMEM((_L,), jnp.float32),
            pltpu.VMEM((_L,), jnp.int32),
            pltpu.VMEM((_TOP,), jnp.float32),
            pltpu.VMEM((_TOP,), jnp.int32),
            pltpu.VMEM_SHARED((_NS * _L,), jnp.int32),
            pltpu.VMEM_SHARED((2 * _L, _NBIN // (2 * _L)), jnp.int32),
            pltpu.VMEM_SHARED((_SLOTS,), jnp.float32),
            pltpu.VMEM_SHARED((_SLOTS,), jnp.int32),
            pltpu.VMEM_SHARED((_SLOTS,), jnp.int32),
        ],
    )
    return fn(_select_body)(prob_p, sub_p, obj_p, scores_p)


def _merge_body(oval_hbm, oidx_hbm, prob_hbm, label_hbm, sub_hbm, obj_hbm,
                conn_hbm, lab_hbm, prb_hbm,
                val_v, idx_v, mv, mi, g1, g2, g3, g4, conn_v, sem):
    c = lax.axis_index("c")
    s = lax.axis_index("s")

    @pl.when((c == 0) & (s == 0))
    def _():
        lane = lax.iota(jnp.int32, _L)
        pltpu.sync_copy(oval_hbm, val_v)
        pltpu.sync_copy(oidx_hbm, idx_v)
        for j in range(_TOP // _L):
            mv[pl.ds(j * _L, _L)] = jnp.full((_L,), -1.0, jnp.float32)
            mi[pl.ds(j * _L, _L)] = jnp.zeros((_L,), jnp.int32)

        for side in range(2):
            for j in range(_TOP // _L):
                pos = j * _L + lane
                va = val_v[pl.ds(side * _TOP + j * _L, _L)]
                ia = idx_v[pl.ds(side * _TOP + j * _L, _L)]
                cnt = jnp.zeros((_L,), jnp.int32)
                for jb in range(_TOP // _L):
                    vb = val_v[pl.ds((1 - side) * _TOP + jb * _L, _L)]
                    ib = idx_v[pl.ds((1 - side) * _TOP + jb * _L, _L)]
                    for q in range(_L):
                        bv = jnp.full((_L,), vb[q], jnp.float32)
                        bi = jnp.full((_L,), ib[q], jnp.int32)
                        beats = (bv > va) | ((bv == va) & (bi < ia))
                        cnt = cnt + beats.astype(jnp.int32)
                rank = pos + cnt
                m = rank < _TOPK
                ridx = jnp.minimum(rank, _TOP - 1)
                plsc.store_scatter(mv, [ridx], va, mask=m)
                plsc.store_scatter(mi, [ridx], ia, mask=m)

        pltpu.async_copy(prob_hbm.at[mi], g1, sem).wait()
        pltpu.async_copy(label_hbm.at[mi], g2, sem).wait()
        pltpu.async_copy(sub_hbm.at[mi], g3, sem).wait()
        pltpu.async_copy(obj_hbm.at[mi], g4, sem).wait()

        for j in range(_TOP // _L):
            pos2 = 2 * (j * _L + lane)
            sub16 = g3[pl.ds(j * _L, _L)]
            obj16 = g4[pl.ds(j * _L, _L)]
            plsc.store_scatter(conn_v, [pos2], sub16)
            plsc.store_scatter(conn_v, [pos2 + 1], obj16)

        pltpu.sync_copy(conn_v, conn_hbm)
        pltpu.sync_copy(g2, lab_hbm)
        pltpu.sync_copy(g1, prb_hbm)


def _sc_merge(oval, oidx, prob, label, sub, obj):
    fn = functools.partial(
        pl.kernel,
        compiler_params=pltpu.CompilerParams(needs_layout_passes=False),
        out_type=[
            jax.ShapeDtypeStruct((2 * _TOP,), jnp.int32),
            jax.ShapeDtypeStruct((_TOP,), jnp.int32),
            jax.ShapeDtypeStruct((_TOP,), jnp.float32),
        ],
        mesh=_sc_mesh(),
        scratch_types=[
            pltpu.VMEM((_NC * _TOP,), jnp.float32),
            pltpu.VMEM((_NC * _TOP,), jnp.int32),
            pltpu.VMEM((_TOP,), jnp.float32),
            pltpu.VMEM((_TOP,), jnp.int32),
            pltpu.VMEM((_TOP,), jnp.float32),
            pltpu.VMEM((_TOP,), jnp.int32),
            pltpu.VMEM((_TOP,), jnp.int32),
            pltpu.VMEM((_TOP,), jnp.int32),
            pltpu.VMEM((2 * _TOP,), jnp.int32),
            pltpu.SemaphoreType.DMA,
        ],
    )
    return fn(_merge_body)(oval, oidx, prob, label, sub, obj)


def kernel(rel_det_prob, scores, connect_arr):
    prob_p, label_p = _maxarg(rel_det_prob)
    pad = _PAD_N - _N_REL
    sub = connect_arr[0]
    obj = connect_arr[1]
    sub_p = jnp.concatenate([sub, jnp.zeros((pad,), jnp.int32)])
    obj_p = jnp.concatenate([obj, jnp.zeros((pad,), jnp.int32)])
    scores_p = jnp.concatenate([scores, jnp.zeros((24,), jnp.float32)])

    oval, oidx = _sc_select(prob_p, sub_p, obj_p, scores_p)
    conn_flat, lab2d, prb2d = _sc_merge(oval, oidx, prob_p, label_p, sub_p,
                                        obj_p)

    conn_sel = conn_flat[: 2 * _TOPK].reshape(_TOPK, 2)
    labels_sel = lab2d[:_TOPK]
    probs_sel = prb2d[:_TOPK]
    return conn_sel, labels_sel, probs_sel
